# baseline (speedup 1.0000x reference)
import functools

import jax
import jax.numpy as jnp
from jax import lax
from jax.experimental import pallas as pl
from jax.experimental.pallas import tpu as pltpu
from jax.experimental.pallas import tpu_sc as plsc

N_NODES = 10000
N_EDGES = 160000
D_FEAT = 256
EPSILON = 0.1
C = 10.0

NC = 2
NS = 16
DH = D_FEAT // NC
E_TILE = N_EDGES // NS
K = 80
CH = E_TILE // K
GRP = 24
NGF = CH // GRP
TAIL = CH - NGF * GRP
TAIL_PAD = 8
CH_PAD = NGF * GRP + TAIL_PAD
NB = 3
SIX = 3
RCH = 80
NRC = N_NODES // RCH
NRC_TILE = (NRC + NS - 1) // NS


def _activation(x):
    mask = (x > EPSILON).astype(x.dtype)
    theta = (x - EPSILON) / (1.0 - EPSILON + 1e-8)
    theta = jnp.clip(theta, 0.0, 1.0)
    numerator = 1.0 + jnp.exp(jnp.asarray(-C, dtype=x.dtype))
    denominator = 1.0 + jnp.exp(-C * theta)
    return mask * (theta * numerator / denominator)


def _mm_act_body(a_ref, w_ref, b_ref, o_ref):
    a = jnp.concatenate([a_ref[0], a_ref[1]], axis=1)
    t = jnp.dot(a, w_ref[...], preferred_element_type=jnp.float32) + b_ref[...]
    o_ref[...] = _activation(t)


def _tc_mm_act(agg, w, b):
    return pl.pallas_call(
        _mm_act_body,
        out_shape=jax.ShapeDtypeStruct((N_NODES, D_FEAT), jnp.float32),
    )(agg, w, b)


def _mm_bias_body(a_ref, w_ref, b_ref, o_ref):
    a = jnp.concatenate([a_ref[0], a_ref[1]], axis=1)
    o_ref[...] = jnp.dot(a, w_ref[...],
                         preferred_element_type=jnp.float32) + b_ref[...]


def _tc_mm_bias(agg, w, b):
    return pl.pallas_call(
        _mm_bias_body,
        out_shape=jax.ShapeDtypeStruct((N_NODES, D_FEAT), jnp.float32),
    )(agg, w, b)


def _agg_body(table_ref, src_ref, dst_ref, zeros_ref, out_ref,
              src_v0, dst_v0, src_v1, dst_v1, rows0, rows1, rows2, acc,
              gsem0, gsem1, gsem2, ssem0, ssem1, ssem2, isem):
    rows = (rows0, rows1, rows2)
    gsem = (gsem0, gsem1, gsem2)
    ssem = (ssem0, ssem1, ssem2)
    sv = (src_v0, src_v1)
    dv = (dst_v0, dst_v1)
    c = lax.axis_index("c")
    s = lax.axis_index("s")
    col = c * DH


    def gather(ib, j, b):
        pltpu.async_copy(table_ref.at[sv[ib].at[j], pl.ds(col, DH)],
                         rows[b], gsem[b])

    def gather_wait(ib, j, b):
        pltpu.make_async_copy(table_ref.at[sv[ib].at[j], pl.ds(col, DH)],
                              rows[b], gsem[b]).wait()

    def scat(ib, j, b):
        pltpu.async_copy(rows[b], acc.at[dv[ib].at[j]], ssem[b], add=True)

    def scat_wait(ib, j, b):
        pltpu.make_async_copy(rows[b], acc.at[dv[ib].at[j]], ssem[b]).wait()

    def refill_async(row0, nrow, ib):
        pltpu.async_copy(src_ref.at[s, pl.ds(row0, nrow)],
                         sv[ib].at[pl.ds(0, nrow)], isem)
        pltpu.async_copy(dst_ref.at[s, pl.ds(row0, nrow)],
                         dv[ib].at[pl.ds(0, nrow)], isem)

    def refill_wait(row0, nrow, ib):
        pltpu.make_async_copy(src_ref.at[s, pl.ds(row0, nrow)],
                              sv[ib].at[pl.ds(0, nrow)], isem).wait()
        pltpu.make_async_copy(dst_ref.at[s, pl.ds(row0, nrow)],
                              dv[ib].at[pl.ds(0, nrow)], isem).wait()

    pltpu.sync_copy(src_ref.at[s, pl.ds(0, GRP)], src_v0)
    pltpu.sync_copy(dst_ref.at[s, pl.ds(0, GRP)], dst_v0)
    gather(0, 0, 0)
    gather(0, 1, 1)

    def zero_body(i, carry):
        j = s * NRC_TILE + i

        @pl.when(j < NRC)
        def _():
            pltpu.sync_copy(zeros_ref, acc.at[pl.ds(j * RCH, RCH)])

        return carry

    lax.fori_loop(0, NRC_TILE, zero_body, 0)
    plsc.subcore_barrier()

    for g in range(NGF):
        ib = g % 2
        nxt_rows = TAIL_PAD if g + 1 == NGF else GRP
        if g == 0:
            refill_async(GRP, GRP, 1)

        def six(t, inner, ib=ib, g=g, nxt_rows=nxt_rows):
            for q in range(SIX):
                j = SIX * t + q
                b = q % NB
                gather_wait(ib, j, b)
                scat(ib, j, b)
                if q == 0:
                    if g > 0:
                        @pl.when(j == 0)
                        def _():
                            scat_wait(1 - ib, GRP - 1, NB - 1)
                            refill_async((g + 1) * GRP, nxt_rows, 1 - ib)

                    @pl.when(j >= 1)
                    def _():
                        scat_wait(ib, j - 1, NB - 1)
                else:
                    scat_wait(ib, j - 1, (q - 1) % NB)

                nb2 = (q + 2) % NB

                @pl.when(j + 2 < GRP)
                def _():
                    gather(ib, j + 2, nb2)

                @pl.when(j + 2 == GRP)
                def _():
                    refill_wait((g + 1) * GRP, nxt_rows, 1 - ib)
                    gather(1 - ib, 0, nb2)

                @pl.when(j + 2 == GRP + 1)
                def _():
                    gather(1 - ib, 1, nb2)
            return inner

        lax.fori_loop(0, GRP // SIX, six, 0)

    tib = NGF % 2
    for j in range(TAIL):
        b = j % NB
        gather_wait(tib, j, b)
        scat(tib, j, b)
        if j == 0:
            scat_wait(1 - tib, GRP - 1, NB - 1)
        else:
            scat_wait(tib, j - 1, (j - 1) % NB)
        if j + 2 < TAIL:
            gather(tib, j + 2, (j + 2) % NB)
    scat_wait(tib, TAIL - 1, (TAIL - 1) % NB)
    plsc.subcore_barrier()

    def out_body(i, carry):
        j = s * NRC_TILE + i

        @pl.when(j < NRC)
        def _():
            pltpu.sync_copy(acc.at[pl.ds(j * RCH, RCH)],
                            out_ref.at[c, pl.ds(j * RCH, RCH)])

        return carry

    lax.fori_loop(0, NRC_TILE, out_body, 0)


_agg_kernel = functools.partial(
    pl.kernel,
    out_type=jax.ShapeDtypeStruct((NC, N_NODES, DH), jnp.float32),
    mesh=plsc.VectorSubcoreMesh(core_axis_name="c", subcore_axis_name="s"),
    scratch_types=[
        pltpu.VMEM((GRP, K), jnp.int32),
        pltpu.VMEM((GRP, K), jnp.int32),
        pltpu.VMEM((GRP, K), jnp.int32),
        pltpu.VMEM((GRP, K), jnp.int32),
        pltpu.VMEM((K, DH), jnp.float32),
        pltpu.VMEM((K, DH), jnp.float32),
        pltpu.VMEM((K, DH), jnp.float32),
        pltpu.VMEM_SHARED((N_NODES, DH), jnp.float32),
        pltpu.SemaphoreType.DMA,
        pltpu.SemaphoreType.DMA,
        pltpu.SemaphoreType.DMA,
        pltpu.SemaphoreType.DMA,
        pltpu.SemaphoreType.DMA,
        pltpu.SemaphoreType.DMA,
        pltpu.SemaphoreType.DMA,
    ],
)(_agg_body)


def _sc_aggregate(table, srcr, dstr, zeros):
    return _agg_kernel(table, srcr, dstr, zeros)


def kernel(x, edge_index, W1, b1, W2, b2):
    src = edge_index[0].astype(jnp.int32)
    dst = edge_index[1].astype(jnp.int32)
    srcr = jnp.pad(src.reshape(NS, CH, K), ((0, 0), (0, CH_PAD - CH), (0, 0)))
    dstr = jnp.pad(dst.reshape(NS, CH, K), ((0, 0), (0, CH_PAD - CH), (0, 0)))
    zeros = jnp.zeros((RCH, DH), jnp.float32)

    a1 = _sc_aggregate(x, srcr, dstr, zeros)
    h = _tc_mm_act(a1, W1, b1)
    a2 = _sc_aggregate(h, srcr, dstr, zeros)
    return _tc_mm_bias(a2, W2, b2)


# speedup vs baseline: 1.1115x; 1.1115x over previous
import functools

import jax
import jax.numpy as jnp
from jax import lax
from jax.experimental import pallas as pl
from jax.experimental.pallas import tpu as pltpu
from jax.experimental.pallas import tpu_sc as plsc

N_NODES = 10000
N_EDGES = 160000
D_FEAT = 256
EPSILON = 0.1
C = 10.0

NC = 2
NS = 16
DH = D_FEAT // NC
E_TILE = N_EDGES // NS
K = 80
CH = E_TILE // K
GRP = 24
NGF = CH // GRP
TAIL = CH - NGF * GRP
TAIL_PAD = 8
CH_PAD = NGF * GRP + TAIL_PAD
NB = 3
SIX = 3
RCH = 80
NRC = N_NODES // RCH
NRC_TILE = (NRC + NS - 1) // NS


def _activation(x):
    mask = (x > EPSILON).astype(x.dtype)
    theta = (x - EPSILON) / (1.0 - EPSILON + 1e-8)
    theta = jnp.clip(theta, 0.0, 1.0)
    numerator = 1.0 + jnp.exp(jnp.asarray(-C, dtype=x.dtype))
    denominator = 1.0 + jnp.exp(-C * theta)
    return mask * (theta * numerator / denominator)


def _mm_act_body(a_ref, w_ref, b_ref, o_ref):
    a = jnp.concatenate([a_ref[0], a_ref[1]], axis=1)
    t = jnp.dot(a, w_ref[...], preferred_element_type=jnp.float32) + b_ref[...]
    o_ref[...] = _activation(t)


def _tc_mm_act(agg, w, b):
    return pl.pallas_call(
        _mm_act_body,
        out_shape=jax.ShapeDtypeStruct((N_NODES, D_FEAT), jnp.float32),
    )(agg, w, b)


def _mm_bias_body(a_ref, w_ref, b_ref, o_ref):
    a = jnp.concatenate([a_ref[0], a_ref[1]], axis=1)
    o_ref[...] = jnp.dot(a, w_ref[...],
                         preferred_element_type=jnp.float32) + b_ref[...]


def _tc_mm_bias(agg, w, b):
    return pl.pallas_call(
        _mm_bias_body,
        out_shape=jax.ShapeDtypeStruct((N_NODES, D_FEAT), jnp.float32),
    )(agg, w, b)


def _agg_body(table_ref, src_ref, dst_ref, out_ref,
              src_v0, dst_v0, src_v1, dst_v1, rows0, rows1, rows2, acc,
              gsem0, gsem1, gsem2, ssem0, ssem1, ssem2, isem):
    rows = (rows0, rows1, rows2)
    gsem = (gsem0, gsem1, gsem2)
    ssem = (ssem0, ssem1, ssem2)
    sv = (src_v0, src_v1)
    dv = (dst_v0, dst_v1)
    c = lax.axis_index("c")
    s = lax.axis_index("s")
    col = c * DH


    def gather(ib, j, b):
        pltpu.async_copy(table_ref.at[sv[ib].at[j], pl.ds(col, DH)],
                         rows[b], gsem[b])

    def gather_wait(ib, j, b):
        pltpu.make_async_copy(table_ref.at[sv[ib].at[j], pl.ds(col, DH)],
                              rows[b], gsem[b]).wait()

    def scat(ib, j, b):
        pltpu.async_copy(rows[b], acc.at[dv[ib].at[j]], ssem[b], add=True)

    def scat_wait(ib, j, b):
        pltpu.make_async_copy(rows[b], acc.at[dv[ib].at[j]], ssem[b]).wait()

    def refill_async(row0, nrow, ib):
        pltpu.async_copy(src_ref.at[s, pl.ds(row0, nrow)],
                         sv[ib].at[pl.ds(0, nrow)], isem)
        pltpu.async_copy(dst_ref.at[s, pl.ds(row0, nrow)],
                         dv[ib].at[pl.ds(0, nrow)], isem)

    def refill_wait(row0, nrow, ib):
        pltpu.make_async_copy(src_ref.at[s, pl.ds(row0, nrow)],
                              sv[ib].at[pl.ds(0, nrow)], isem).wait()
        pltpu.make_async_copy(dst_ref.at[s, pl.ds(row0, nrow)],
                              dv[ib].at[pl.ds(0, nrow)], isem).wait()

    pltpu.sync_copy(src_ref.at[s, pl.ds(0, GRP)], src_v0)
    pltpu.sync_copy(dst_ref.at[s, pl.ds(0, GRP)], dst_v0)
    gather(0, 0, 0)
    gather(0, 1, 1)

    zvec = jnp.zeros((16,), jnp.float32)

    def zfill(i, carry):
        r = i // (DH // 16)
        v = i % (DH // 16)
        rows2[r, pl.ds(v * 16, 16)] = zvec
        return carry

    lax.fori_loop(0, RCH * (DH // 16), zfill, 0)

    def zero_body(i, carry):
        j = s * NRC_TILE + i

        @pl.when(j < NRC)
        def _():
            pltpu.sync_copy(rows2.at[pl.ds(0, RCH)], acc.at[pl.ds(j * RCH, RCH)])

        return carry

    lax.fori_loop(0, NRC_TILE, zero_body, 0)
    plsc.subcore_barrier()

    for g in range(NGF):
        ib = g % 2
        nxt_rows = TAIL_PAD if g + 1 == NGF else GRP
        if g == 0:
            refill_async(GRP, GRP, 1)

        def six(t, inner, ib=ib, g=g, nxt_rows=nxt_rows):
            for q in range(SIX):
                j = SIX * t + q
                b = q % NB
                gather_wait(ib, j, b)
                scat(ib, j, b)
                if q == 0:
                    if g > 0:
                        @pl.when(j == 0)
                        def _():
                            scat_wait(1 - ib, GRP - 1, NB - 1)
                            refill_async((g + 1) * GRP, nxt_rows, 1 - ib)

                    @pl.when(j >= 1)
                    def _():
                        scat_wait(ib, j - 1, NB - 1)
                else:
                    scat_wait(ib, j - 1, (q - 1) % NB)

                nb2 = (q + 2) % NB

                @pl.when(j + 2 < GRP)
                def _():
                    gather(ib, j + 2, nb2)

                @pl.when(j + 2 == GRP)
                def _():
                    refill_wait((g + 1) * GRP, nxt_rows, 1 - ib)
                    gather(1 - ib, 0, nb2)

                @pl.when(j + 2 == GRP + 1)
                def _():
                    gather(1 - ib, 1, nb2)
            return inner

        lax.fori_loop(0, GRP // SIX, six, 0)

    tib = NGF % 2
    for j in range(TAIL):
        b = j % NB
        gather_wait(tib, j, b)
        scat(tib, j, b)
        if j == 0:
            scat_wait(1 - tib, GRP - 1, NB - 1)
        else:
            scat_wait(tib, j - 1, (j - 1) % NB)
        if j + 2 < TAIL:
            gather(tib, j + 2, (j + 2) % NB)
    scat_wait(tib, TAIL - 1, (TAIL - 1) % NB)
    plsc.subcore_barrier()

    def out_body(i, carry):
        j = s * NRC_TILE + i

        @pl.when(j < NRC)
        def _():
            pltpu.sync_copy(acc.at[pl.ds(j * RCH, RCH)],
                            out_ref.at[c, pl.ds(j * RCH, RCH)])

        return carry

    lax.fori_loop(0, NRC_TILE, out_body, 0)


_agg_kernel = functools.partial(
    pl.kernel,
    out_type=jax.ShapeDtypeStruct((NC, N_NODES, DH), jnp.float32),
    mesh=plsc.VectorSubcoreMesh(core_axis_name="c", subcore_axis_name="s"),
    scratch_types=[
        pltpu.VMEM((GRP, K), jnp.int32),
        pltpu.VMEM((GRP, K), jnp.int32),
        pltpu.VMEM((GRP, K), jnp.int32),
        pltpu.VMEM((GRP, K), jnp.int32),
        pltpu.VMEM((K, DH), jnp.float32),
        pltpu.VMEM((K, DH), jnp.float32),
        pltpu.VMEM((K, DH), jnp.float32),
        pltpu.VMEM_SHARED((N_NODES, DH), jnp.float32),
        pltpu.SemaphoreType.DMA,
        pltpu.SemaphoreType.DMA,
        pltpu.SemaphoreType.DMA,
        pltpu.SemaphoreType.DMA,
        pltpu.SemaphoreType.DMA,
        pltpu.SemaphoreType.DMA,
        pltpu.SemaphoreType.DMA,
    ],
)(_agg_body)


def _sc_aggregate(table, srcr, dstr):
    return _agg_kernel(table, srcr, dstr)


def kernel(x, edge_index, W1, b1, W2, b2):
    src = edge_index[0].astype(jnp.int32)
    dst = edge_index[1].astype(jnp.int32)
    srcr = jnp.pad(src.reshape(NS, CH, K), ((0, 0), (0, CH_PAD - CH), (0, 0)))
    dstr = jnp.pad(dst.reshape(NS, CH, K), ((0, 0), (0, CH_PAD - CH), (0, 0)))

    a1 = _sc_aggregate(x, srcr, dstr)
    h = _tc_mm_act(a1, W1, b1)
    a2 = _sc_aggregate(h, srcr, dstr)
    return _tc_mm_bias(a2, W2, b2)
